# Initial kernel scaffold; baseline (speedup 1.0000x reference)
import jax
import jax.numpy as jnp
from jax.experimental import pallas as pl


def kernel(x, stem_w, stem_b, down0_w, down0_b, down0_gamma, down0_beta, down1_w, down1_b, down1_gamma, down1_beta, down2_w, down2_b, down2_gamma, down2_beta, fmtl0_w, fmtl0_b, fmtl1_w, fmtl1_b, fmtl2_w, fmtl2_b, fc_w, fc_b, const, block0_w, block0_b, block1_w, block1_b, block2_w, block2_b, block3_w, block3_b, torgb_w, torgb_b, style_w, style_b):
    raise NotImplementedError("write your pallas kernel here")


# speedup vs baseline: 7.2219x; 7.2219x over previous
import functools

import jax
import jax.numpy as jnp
from jax.experimental import pallas as pl
from jax.experimental.pallas import tpu as pltpu

_VMEM = 64 * 1024 * 1024
_SQRT2 = 1.4142135623730951
_TAPS = tuple((kh, kw) for kh in range(3) for kw in range(3))


def _mish(y):
    t = jnp.exp(jnp.minimum(y, 20.0))
    u = 1.0 + t
    n = u * u
    return y * (n - 1.0) * pl.reciprocal(n + 1.0, approx=True)


def _lrelu(y):
    return jnp.where(y >= 0.0, y, 0.2 * y) * _SQRT2


def _store_padded(ref, h, hw):
    z = jnp.zeros((1, hw + 2, h.shape[-1]), h.dtype)
    ref[pl.ds(0, 1), pl.ds(0, hw + 2), :] = z
    ref[pl.ds(hw + 1, 1), pl.ds(0, hw + 2), :] = z
    zc = jnp.zeros((hw, 1, h.shape[-1]), h.dtype)
    ref[pl.ds(1, hw), pl.ds(0, 1), :] = zc
    ref[pl.ds(1, hw), pl.ds(hw + 1, 1), :] = zc
    ref[pl.ds(1, hw), pl.ds(1, hw), :] = h


def _store_padded_up2(ref, xm, hw_out):
    hw_in = hw_out // 2
    z = jnp.zeros((1, hw_out + 2, xm.shape[-1]), xm.dtype)
    ref[pl.ds(0, 1), pl.ds(0, hw_out + 2), :] = z
    ref[pl.ds(hw_out + 1, 1), pl.ds(0, hw_out + 2), :] = z
    zc = jnp.zeros((hw_out, 1, xm.shape[-1]), xm.dtype)
    ref[pl.ds(1, hw_out), pl.ds(0, 1), :] = zc
    ref[pl.ds(1, hw_out), pl.ds(hw_out + 1, 1), :] = zc
    for a in (0, 1):
        for b in (0, 1):
            ref[pl.Slice(1 + a, hw_in, 2), pl.Slice(1 + b, hw_in, 2), :] = xm


def _conv_s1(pref, wt, ho, wo):
    m = ho * wo
    acc = jnp.zeros((m, wt.shape[-1]), jnp.float32)
    for t, (kh, kw) in enumerate(_TAPS):
        xs = pref[pl.ds(kh, ho), pl.ds(kw, wo), :]
        acc = acc + jnp.dot(xs.reshape(m, xs.shape[-1]), wt[t],
                            preferred_element_type=jnp.float32)
    return acc


def _conv_s2(pref, wt, ho, wo):
    m = ho * wo
    acc = jnp.zeros((m, wt.shape[-1]), jnp.float32)
    for t, (kh, kw) in enumerate(_TAPS):
        xs = pref[pl.Slice(kh, ho, 2), pl.Slice(kw, wo, 2), :]
        xs = xs.astype(jnp.bfloat16).reshape(m, xs.shape[-1])
        acc = acc + jnp.dot(xs, wt[t], preferred_element_type=jnp.float32)
    return acc


def _k1(x_ref, sw_ref, sb_ref, dw_ref, db_ref, y_ref, st_ref, pad_ref):
    acc = jnp.zeros((4096, 128), jnp.float32)
    for t, (kh, kw) in enumerate(_TAPS):
        xs = x_ref[0, pl.ds(kh, 64), pl.ds(kw, 64), :].reshape(4096, 3)
        acc = acc + jnp.dot(xs, sw_ref[t], preferred_element_type=jnp.float32)
    h = _mish(acc + sb_ref[...]).astype(jnp.bfloat16).reshape(64, 64, 128)
    _store_padded(pad_ref, h.astype(jnp.float32), 64)
    y = _conv_s2(pad_ref, dw_ref[...], 32, 32) + db_ref[...]
    st_ref[0, 0:1, :] = jnp.sum(y, axis=0, keepdims=True)
    st_ref[0, 1:2, :] = jnp.sum(y * y, axis=0, keepdims=True)
    y_ref[0] = y.reshape(32, 32, 128).astype(jnp.bfloat16)


def _k23(a_ref, sc_ref, sh_ref, w_ref, b_ref, y_ref, st_ref, pad_ref, *, hi):
    ho = hi // 2
    h = _mish(a_ref[0].astype(jnp.float32) * sc_ref[...] + sh_ref[...])
    _store_padded(pad_ref, h.astype(jnp.bfloat16).astype(jnp.float32), hi)
    y = _conv_s2(pad_ref, w_ref[...], ho, ho) + b_ref[...]
    st_ref[0, 0:1, :] = jnp.sum(y, axis=0, keepdims=True)
    st_ref[0, 1:2, :] = jnp.sum(y * y, axis=0, keepdims=True)
    y_ref[0] = y.reshape(ho, ho, 128).astype(jnp.bfloat16)


def _k4(a_ref, sc_ref, sh_ref, w0_ref, b0_ref, w1_ref, b1_ref,
        w2_ref, b2_ref, fw_ref, fb_ref, z_ref, p0_ref, p1_ref, p2_ref):
    h = _mish(a_ref[0].astype(jnp.float32) * sc_ref[...] + sh_ref[...])
    _store_padded(p0_ref, h.astype(jnp.bfloat16).astype(jnp.float32), 8)
    h = _mish(_conv_s2(p0_ref, w0_ref[...], 4, 4) + b0_ref[...])
    h = h.astype(jnp.bfloat16).astype(jnp.float32).reshape(4, 4, 128)
    _store_padded(p1_ref, h, 4)
    h = _mish(_conv_s2(p1_ref, w1_ref[...], 2, 2) + b1_ref[...])
    h = h.astype(jnp.bfloat16).astype(jnp.float32).reshape(2, 2, 128)
    _store_padded(p2_ref, h, 2)
    h = _mish(_conv_s2(p2_ref, w2_ref[...], 1, 1) + b2_ref[...])
    z_ref[0] = jnp.dot(h.astype(jnp.bfloat16), fw_ref[...],
                       preferred_element_type=jnp.float32) + fb_ref[...]


def _k5(z_ref, sw_ref, sb_ref, q_ref, s_ref, d_ref):
    st = jnp.dot(z_ref[...], sw_ref[0],
                 preferred_element_type=jnp.float32) + sb_ref[0]
    s_ref[0] = st
    d_ref[0] = jax.lax.rsqrt(
        jnp.dot(st * st, q_ref[0], preferred_element_type=jnp.float32) + 1e-8)


def _k6(c_ref, st_ref, d_ref, wa_ref, wb_ref, wc_ref, wd_ref,
        bias_ref, tw_ref, tb_ref, o_ref, pa_ref,
        qb_ref, pb_ref, qc_ref, pc_ref, qd_ref, pd_ref):
    st = st_ref[0]
    dm = d_ref[0]
    x = c_ref[...].astype(jnp.bfloat16).astype(jnp.float32) * st[0:1, :]
    _store_padded(pa_ref, x.astype(jnp.bfloat16), 8)
    ws = (wa_ref, wb_ref, wc_ref, wd_ref)
    pads = (pa_ref, pb_ref, pc_ref, pd_ref)
    ups = (None, qb_ref, qc_ref, qd_ref)
    for g, hw in enumerate((8, 16, 32, 64)):
        if g > 0:
            xm = (x.astype(jnp.float32) * st[g:g + 1, :]).astype(jnp.bfloat16)
            _store_padded_up2(ups[g], xm.astype(jnp.float32), hw)
            pads[g][...] = ups[g][...].astype(jnp.bfloat16)
        y = _conv_s1(pads[g], ws[g][...], hw, hw)
        y = y * dm[g:g + 1, :] + bias_ref[g]
        x = _lrelu(y).astype(jnp.bfloat16).reshape(hw, hw, 128)
    xm = (x.astype(jnp.float32) * st[4:5, :]).astype(jnp.bfloat16)
    y = jnp.dot(xm.reshape(4096, 128), tw_ref[...],
                preferred_element_type=jnp.float32) + tb_ref[...]
    o_ref[0] = y[:, :8].astype(jnp.bfloat16).reshape(64, 64, 8)


def _bn_coeffs(stats, gamma, beta, cnt):
    tot = jnp.sum(stats, axis=0)
    mean = tot[0] / cnt
    var = jnp.maximum(tot[1] / cnt - mean * mean, 0.0)
    inv = jax.lax.rsqrt(var + 1e-5)
    sc = (gamma * inv).astype(jnp.float32)
    sh = (beta - mean * sc).astype(jnp.float32)
    return sc.reshape(1, 128), sh.reshape(1, 128)


def _cparams(sem):
    return pltpu.CompilerParams(dimension_semantics=sem,
                                vmem_limit_bytes=_VMEM)


def kernel(x, stem_w, stem_b, down0_w, down0_b, down0_gamma, down0_beta,
           down1_w, down1_b, down1_gamma, down1_beta,
           down2_w, down2_b, down2_gamma, down2_beta,
           fmtl0_w, fmtl0_b, fmtl1_w, fmtl1_b, fmtl2_w, fmtl2_b,
           fc_w, fc_b, const,
           block0_w, block0_b, block1_w, block1_b,
           block2_w, block2_b, block3_w, block3_b,
           torgb_w, torgb_b, style_w, style_b):
    f32, bf16 = jnp.float32, jnp.bfloat16
    N = x.shape[0]

    xt = jnp.transpose(x, (0, 2, 3, 1)).astype(bf16)
    xt = jnp.pad(xt, ((0, 0), (1, 1), (1, 1), (0, 0)))

    sw = stem_w.reshape(9, 3, 128).astype(bf16)
    sb = stem_b.reshape(1, 128).astype(f32)
    d0w = down0_w.reshape(9, 128, 128).astype(bf16)
    d0b = down0_b.reshape(1, 128).astype(f32)

    a0, s0 = pl.pallas_call(
        _k1,
        grid=(N,),
        in_specs=[
            pl.BlockSpec((1, 66, 66, 3), lambda n: (n, 0, 0, 0)),
            pl.BlockSpec((9, 3, 128), lambda n: (0, 0, 0)),
            pl.BlockSpec((1, 128), lambda n: (0, 0)),
            pl.BlockSpec((9, 128, 128), lambda n: (0, 0, 0)),
            pl.BlockSpec((1, 128), lambda n: (0, 0)),
        ],
        out_specs=(pl.BlockSpec((1, 32, 32, 128), lambda n: (n, 0, 0, 0)),
                   pl.BlockSpec((1, 2, 128), lambda n: (n, 0, 0))),
        out_shape=(jax.ShapeDtypeStruct((N, 32, 32, 128), bf16),
                   jax.ShapeDtypeStruct((N, 2, 128), f32)),
        scratch_shapes=[pltpu.VMEM((66, 66, 128), f32)],
        compiler_params=_cparams(("parallel",)),
    )(xt, sw, sb, d0w, d0b)

    def down_call(a, stats, gamma, beta, w, b, hi):
        cnt = float(N * hi * hi)
        sc, sh = _bn_coeffs(stats, gamma, beta, cnt)
        ho = hi // 2
        return pl.pallas_call(
            functools.partial(_k23, hi=hi),
            grid=(N,),
            in_specs=[
                pl.BlockSpec((1, hi, hi, 128), lambda n: (n, 0, 0, 0)),
                pl.BlockSpec((1, 128), lambda n: (0, 0)),
                pl.BlockSpec((1, 128), lambda n: (0, 0)),
                pl.BlockSpec((9, 128, 128), lambda n: (0, 0, 0)),
                pl.BlockSpec((1, 128), lambda n: (0, 0)),
            ],
            out_specs=(pl.BlockSpec((1, ho, ho, 128), lambda n: (n, 0, 0, 0)),
                       pl.BlockSpec((1, 2, 128), lambda n: (n, 0, 0))),
            out_shape=(jax.ShapeDtypeStruct((N, ho, ho, 128), bf16),
                       jax.ShapeDtypeStruct((N, 2, 128), f32)),
            scratch_shapes=[pltpu.VMEM((hi + 2, hi + 2, 128), f32)],
            compiler_params=_cparams(("parallel",)),
        )(a, sc, sh, w.reshape(9, 128, 128).astype(bf16),
          b.reshape(1, 128).astype(f32))

    a1, s1 = down_call(a0, s0, down0_gamma, down0_beta, down1_w, down1_b, 32)
    a2, s2 = down_call(a1, s1, down1_gamma, down1_beta, down2_w, down2_b, 16)

    sc2, sh2 = _bn_coeffs(s2, down2_gamma, down2_beta, float(N * 8 * 8))
    z = pl.pallas_call(
        _k4,
        grid=(N,),
        in_specs=[
            pl.BlockSpec((1, 8, 8, 128), lambda n: (n, 0, 0, 0)),
            pl.BlockSpec((1, 128), lambda n: (0, 0)),
            pl.BlockSpec((1, 128), lambda n: (0, 0)),
            pl.BlockSpec((9, 128, 128), lambda n: (0, 0, 0)),
            pl.BlockSpec((1, 128), lambda n: (0, 0)),
            pl.BlockSpec((9, 128, 128), lambda n: (0, 0, 0)),
            pl.BlockSpec((1, 128), lambda n: (0, 0)),
            pl.BlockSpec((9, 128, 128), lambda n: (0, 0, 0)),
            pl.BlockSpec((1, 128), lambda n: (0, 0)),
            pl.BlockSpec((128, 128), lambda n: (0, 0)),
            pl.BlockSpec((1, 128), lambda n: (0, 0)),
        ],
        out_specs=pl.BlockSpec((1, 1, 128), lambda n: (n, 0, 0)),
        out_shape=jax.ShapeDtypeStruct((N, 1, 128), f32),
        scratch_shapes=[pltpu.VMEM((10, 10, 128), f32),
                        pltpu.VMEM((6, 6, 128), f32),
                        pltpu.VMEM((4, 4, 128), f32)],
        compiler_params=_cparams(("parallel",)),
    )(a2, sc2, sh2,
      fmtl0_w.reshape(9, 128, 128).astype(bf16), fmtl0_b.reshape(1, 128).astype(f32),
      fmtl1_w.reshape(9, 128, 128).astype(bf16), fmtl1_b.reshape(1, 128).astype(f32),
      fmtl2_w.reshape(9, 128, 128).astype(bf16), fmtl2_b.reshape(1, 128).astype(f32),
      fc_w.reshape(128, 128).astype(bf16), fc_b.reshape(1, 128).astype(f32))

    zb = z.reshape(N, 128).astype(bf16)
    bws = (block0_w, block1_w, block2_w, block3_w)
    wsq = jnp.stack([jnp.sum(w * w, axis=(0, 1)) for w in bws])
    wsq = jnp.concatenate([wsq, jnp.zeros((1, 128, 128), f32)], axis=0)

    styles5, d5 = pl.pallas_call(
        _k5,
        grid=(5,),
        in_specs=[
            pl.BlockSpec((N, 128), lambda g: (0, 0)),
            pl.BlockSpec((1, 128, 128), lambda g: (g, 0, 0)),
            pl.BlockSpec((1, 1, 128), lambda g: (g, 0, 0)),
            pl.BlockSpec((1, 128, 128), lambda g: (g, 0, 0)),
        ],
        out_specs=(pl.BlockSpec((1, N, 128), lambda g: (g, 0, 0)),
                   pl.BlockSpec((1, N, 128), lambda g: (g, 0, 0))),
        out_shape=(jax.ShapeDtypeStruct((5, N, 128), f32),
                   jax.ShapeDtypeStruct((5, N, 128), f32)),
        compiler_params=_cparams(("parallel",)),
    )(zb, style_w.astype(bf16), style_b.astype(f32), wsq)

    stp = jnp.transpose(styles5, (1, 0, 2))
    dp = jnp.transpose(d5[:4], (1, 0, 2))

    bias_dec = jnp.stack([block0_b, block1_b, block2_b, block3_b])
    bias_dec = bias_dec.reshape(4, 1, 128).astype(f32)

    out = pl.pallas_call(
        _k6,
        grid=(N,),
        in_specs=[
            pl.BlockSpec((8, 8, 128), lambda n: (0, 0, 0)),
            pl.BlockSpec((1, 5, 128), lambda n: (n, 0, 0)),
            pl.BlockSpec((1, 4, 128), lambda n: (n, 0, 0)),
            pl.BlockSpec((9, 128, 128), lambda n: (0, 0, 0)),
            pl.BlockSpec((9, 128, 128), lambda n: (0, 0, 0)),
            pl.BlockSpec((9, 128, 128), lambda n: (0, 0, 0)),
            pl.BlockSpec((9, 128, 128), lambda n: (0, 0, 0)),
            pl.BlockSpec((4, 1, 128), lambda n: (0, 0, 0)),
            pl.BlockSpec((128, 128), lambda n: (0, 0)),
            pl.BlockSpec((1, 128), lambda n: (0, 0)),
        ],
        out_specs=pl.BlockSpec((1, 64, 64, 8), lambda n: (n, 0, 0, 0)),
        out_shape=jax.ShapeDtypeStruct((N, 64, 64, 8), bf16),
        scratch_shapes=[pltpu.VMEM((10, 10, 128), bf16),
                        pltpu.VMEM((18, 18, 128), f32),
                        pltpu.VMEM((18, 18, 128), bf16),
                        pltpu.VMEM((34, 34, 128), f32),
                        pltpu.VMEM((34, 34, 128), bf16),
                        pltpu.VMEM((66, 66, 128), f32),
                        pltpu.VMEM((66, 66, 128), bf16)],
        compiler_params=_cparams(("parallel",)),
    )(const, stp, dp,
      block0_w.reshape(9, 128, 128).astype(bf16),
      block1_w.reshape(9, 128, 128).astype(bf16),
      block2_w.reshape(9, 128, 128).astype(bf16),
      block3_w.reshape(9, 128, 128).astype(bf16),
      bias_dec,
      torgb_w.reshape(128, 128).astype(bf16),
      torgb_b.reshape(1, 128).astype(f32))

    y = out[..., :3].astype(f32)
    return jnp.transpose(y, (0, 3, 1, 2))


# speedup vs baseline: 9.6774x; 1.3400x over previous
import functools

import jax
import jax.numpy as jnp
from jax.experimental import pallas as pl
from jax.experimental.pallas import tpu as pltpu

_VMEM = 64 * 1024 * 1024
_SQRT2 = 1.4142135623730951
_TAPS = tuple((kh, kw) for kh in range(3) for kw in range(3))


def _mish(y):
    t = jnp.exp(jnp.minimum(y, 20.0))
    u = 1.0 + t
    n = u * u
    return y * (n - 1.0) * pl.reciprocal(n + 1.0, approx=True)


def _lrelu(y):
    return jnp.where(y >= 0.0, y, 0.2 * y) * _SQRT2


def _store_padded(ref, h, hw):
    z = jnp.zeros((1, hw + 2, h.shape[-1]), h.dtype)
    ref[pl.ds(0, 1), pl.ds(0, hw + 2), :] = z
    ref[pl.ds(hw + 1, 1), pl.ds(0, hw + 2), :] = z
    zc = jnp.zeros((hw, 1, h.shape[-1]), h.dtype)
    ref[pl.ds(1, hw), pl.ds(0, 1), :] = zc
    ref[pl.ds(1, hw), pl.ds(hw + 1, 1), :] = zc
    ref[pl.ds(1, hw), pl.ds(1, hw), :] = h


def _store_padded_up2(ref, xm, hw_out):
    hw_in = hw_out // 2
    z = jnp.zeros((1, hw_out + 2, xm.shape[-1]), xm.dtype)
    ref[pl.ds(0, 1), pl.ds(0, hw_out + 2), :] = z
    ref[pl.ds(hw_out + 1, 1), pl.ds(0, hw_out + 2), :] = z
    zc = jnp.zeros((hw_out, 1, xm.shape[-1]), xm.dtype)
    ref[pl.ds(1, hw_out), pl.ds(0, 1), :] = zc
    ref[pl.ds(1, hw_out), pl.ds(hw_out + 1, 1), :] = zc
    for a in (0, 1):
        for b in (0, 1):
            ref[pl.Slice(1 + a, hw_in, 2), pl.Slice(1 + b, hw_in, 2), :] = xm


def _conv_s1(pref, wt, ho, wo):
    m = ho * wo
    acc = jnp.zeros((m, wt.shape[-1]), jnp.float32)
    for t, (kh, kw) in enumerate(_TAPS):
        xs = pref[pl.ds(kh, ho), pl.ds(kw, wo), :]
        acc = acc + jnp.dot(xs.reshape(m, xs.shape[-1]), wt[t],
                            preferred_element_type=jnp.float32)
    return acc


def _conv_s2(pref, wt, ho, wo):
    m = ho * wo
    acc = jnp.zeros((m, wt.shape[-1]), jnp.float32)
    for t, (kh, kw) in enumerate(_TAPS):
        xs = pref[pl.Slice(kh, ho, 2), pl.Slice(kw, wo, 2), :]
        xs = xs.astype(jnp.bfloat16).reshape(m, xs.shape[-1])
        acc = acc + jnp.dot(xs, wt[t], preferred_element_type=jnp.float32)
    return acc


def _k1(x_ref, sw_ref, sb_ref, dw_ref, db_ref, y_ref, st_ref, xp_ref, pad_ref):
    xb = jnp.transpose(x_ref[0].astype(jnp.bfloat16), (1, 2, 0))
    _store_padded(xp_ref, xb, 64)
    acc = jnp.zeros((4096, 128), jnp.float32)
    for t, (kh, kw) in enumerate(_TAPS):
        xs = xp_ref[pl.ds(kh, 64), pl.ds(kw, 64), :].reshape(4096, 3)
        acc = acc + jnp.dot(xs, sw_ref[t], preferred_element_type=jnp.float32)
    h = _mish(acc + sb_ref[...]).astype(jnp.bfloat16).reshape(64, 64, 128)
    _store_padded(pad_ref, h.astype(jnp.float32), 64)
    y = _conv_s2(pad_ref, dw_ref[...], 32, 32) + db_ref[...]
    st_ref[0, 0:1, :] = jnp.sum(y, axis=0, keepdims=True)
    st_ref[0, 1:2, :] = jnp.sum(y * y, axis=0, keepdims=True)
    y_ref[0] = y.reshape(32, 32, 128).astype(jnp.bfloat16)


def _k23(a_ref, sc_ref, sh_ref, w_ref, b_ref, y_ref, st_ref, pad_ref, *, hi):
    ho = hi // 2
    h = _mish(a_ref[0].astype(jnp.float32) * sc_ref[...] + sh_ref[...])
    _store_padded(pad_ref, h.astype(jnp.bfloat16).astype(jnp.float32), hi)
    y = _conv_s2(pad_ref, w_ref[...], ho, ho) + b_ref[...]
    st_ref[0, 0:1, :] = jnp.sum(y, axis=0, keepdims=True)
    st_ref[0, 1:2, :] = jnp.sum(y * y, axis=0, keepdims=True)
    y_ref[0] = y.reshape(ho, ho, 128).astype(jnp.bfloat16)


def _k4(a_ref, sc_ref, sh_ref, w0_ref, b0_ref, w1_ref, b1_ref,
        w2_ref, b2_ref, fw_ref, fb_ref, z_ref, p0_ref, p1_ref, p2_ref):
    h = _mish(a_ref[0].astype(jnp.float32) * sc_ref[...] + sh_ref[...])
    _store_padded(p0_ref, h.astype(jnp.bfloat16).astype(jnp.float32), 8)
    h = _mish(_conv_s2(p0_ref, w0_ref[...], 4, 4) + b0_ref[...])
    h = h.astype(jnp.bfloat16).astype(jnp.float32).reshape(4, 4, 128)
    _store_padded(p1_ref, h, 4)
    h = _mish(_conv_s2(p1_ref, w1_ref[...], 2, 2) + b1_ref[...])
    h = h.astype(jnp.bfloat16).astype(jnp.float32).reshape(2, 2, 128)
    _store_padded(p2_ref, h, 2)
    h = _mish(_conv_s2(p2_ref, w2_ref[...], 1, 1) + b2_ref[...])
    z_ref[0] = jnp.dot(h.astype(jnp.bfloat16), fw_ref[...],
                       preferred_element_type=jnp.float32) + fb_ref[...]


def _k5(z_ref, sw_ref, sb_ref, q_ref, s_ref, d_ref):
    st = jnp.dot(z_ref[...], sw_ref[0],
                 preferred_element_type=jnp.float32) + sb_ref[0]
    s_ref[0] = st
    d_ref[0] = jax.lax.rsqrt(
        jnp.dot(st * st, q_ref[0], preferred_element_type=jnp.float32) + 1e-8)


def _k6(c_ref, st_ref, d_ref, wa_ref, wb_ref, wc_ref, wd_ref,
        bias_ref, tw_ref, tb_ref, o_ref, pa_ref,
        qb_ref, pb_ref, qc_ref, pc_ref, qd_ref, pd_ref):
    st = st_ref[0]
    dm = d_ref[0]
    x = c_ref[...].astype(jnp.bfloat16).astype(jnp.float32) * st[0:1, :]
    _store_padded(pa_ref, x.astype(jnp.bfloat16), 8)
    ws = (wa_ref, wb_ref, wc_ref, wd_ref)
    pads = (pa_ref, pb_ref, pc_ref, pd_ref)
    ups = (None, qb_ref, qc_ref, qd_ref)
    for g, hw in enumerate((8, 16, 32, 64)):
        if g > 0:
            xm = (x.astype(jnp.float32) * st[g:g + 1, :]).astype(jnp.bfloat16)
            _store_padded_up2(ups[g], xm.astype(jnp.float32), hw)
            pads[g][...] = ups[g][...].astype(jnp.bfloat16)
        y = _conv_s1(pads[g], ws[g][...], hw, hw)
        y = y * dm[g:g + 1, :] + bias_ref[g]
        x = _lrelu(y).astype(jnp.bfloat16).reshape(hw, hw, 128)
    xm = (x.astype(jnp.float32) * st[4:5, :]).astype(jnp.bfloat16)
    y = jnp.dot(xm.reshape(4096, 128), tw_ref[...],
                preferred_element_type=jnp.float32) + tb_ref[...]
    y3 = y[:, :3].astype(jnp.bfloat16).astype(jnp.float32)
    o_ref[0] = jnp.transpose(y3.reshape(64, 64, 3), (2, 0, 1))


def _bn_coeffs(stats, gamma, beta, cnt):
    tot = jnp.sum(stats, axis=0)
    mean = tot[0] / cnt
    var = jnp.maximum(tot[1] / cnt - mean * mean, 0.0)
    inv = jax.lax.rsqrt(var + 1e-5)
    sc = (gamma * inv).astype(jnp.float32)
    sh = (beta - mean * sc).astype(jnp.float32)
    return sc.reshape(1, 128), sh.reshape(1, 128)


def _cparams(sem):
    return pltpu.CompilerParams(dimension_semantics=sem,
                                vmem_limit_bytes=_VMEM)


def kernel(x, stem_w, stem_b, down0_w, down0_b, down0_gamma, down0_beta,
           down1_w, down1_b, down1_gamma, down1_beta,
           down2_w, down2_b, down2_gamma, down2_beta,
           fmtl0_w, fmtl0_b, fmtl1_w, fmtl1_b, fmtl2_w, fmtl2_b,
           fc_w, fc_b, const,
           block0_w, block0_b, block1_w, block1_b,
           block2_w, block2_b, block3_w, block3_b,
           torgb_w, torgb_b, style_w, style_b):
    f32, bf16 = jnp.float32, jnp.bfloat16
    N = x.shape[0]

    sw = stem_w.reshape(9, 3, 128).astype(bf16)
    sb = stem_b.reshape(1, 128).astype(f32)
    d0w = down0_w.reshape(9, 128, 128).astype(bf16)
    d0b = down0_b.reshape(1, 128).astype(f32)

    a0, s0 = pl.pallas_call(
        _k1,
        grid=(N,),
        in_specs=[
            pl.BlockSpec((1, 3, 64, 64), lambda n: (n, 0, 0, 0)),
            pl.BlockSpec((9, 3, 128), lambda n: (0, 0, 0)),
            pl.BlockSpec((1, 128), lambda n: (0, 0)),
            pl.BlockSpec((9, 128, 128), lambda n: (0, 0, 0)),
            pl.BlockSpec((1, 128), lambda n: (0, 0)),
        ],
        out_specs=(pl.BlockSpec((1, 32, 32, 128), lambda n: (n, 0, 0, 0)),
                   pl.BlockSpec((1, 2, 128), lambda n: (n, 0, 0))),
        out_shape=(jax.ShapeDtypeStruct((N, 32, 32, 128), bf16),
                   jax.ShapeDtypeStruct((N, 2, 128), f32)),
        scratch_shapes=[pltpu.VMEM((66, 66, 3), bf16),
                        pltpu.VMEM((66, 66, 128), f32)],
        compiler_params=_cparams(("parallel",)),
    )(x, sw, sb, d0w, d0b)

    def down_call(a, stats, gamma, beta, w, b, hi):
        cnt = float(N * hi * hi)
        sc, sh = _bn_coeffs(stats, gamma, beta, cnt)
        ho = hi // 2
        return pl.pallas_call(
            functools.partial(_k23, hi=hi),
            grid=(N,),
            in_specs=[
                pl.BlockSpec((1, hi, hi, 128), lambda n: (n, 0, 0, 0)),
                pl.BlockSpec((1, 128), lambda n: (0, 0)),
                pl.BlockSpec((1, 128), lambda n: (0, 0)),
                pl.BlockSpec((9, 128, 128), lambda n: (0, 0, 0)),
                pl.BlockSpec((1, 128), lambda n: (0, 0)),
            ],
            out_specs=(pl.BlockSpec((1, ho, ho, 128), lambda n: (n, 0, 0, 0)),
                       pl.BlockSpec((1, 2, 128), lambda n: (n, 0, 0))),
            out_shape=(jax.ShapeDtypeStruct((N, ho, ho, 128), bf16),
                       jax.ShapeDtypeStruct((N, 2, 128), f32)),
            scratch_shapes=[pltpu.VMEM((hi + 2, hi + 2, 128), f32)],
            compiler_params=_cparams(("parallel",)),
        )(a, sc, sh, w.reshape(9, 128, 128).astype(bf16),
          b.reshape(1, 128).astype(f32))

    a1, s1 = down_call(a0, s0, down0_gamma, down0_beta, down1_w, down1_b, 32)
    a2, s2 = down_call(a1, s1, down1_gamma, down1_beta, down2_w, down2_b, 16)

    sc2, sh2 = _bn_coeffs(s2, down2_gamma, down2_beta, float(N * 8 * 8))
    z = pl.pallas_call(
        _k4,
        grid=(N,),
        in_specs=[
            pl.BlockSpec((1, 8, 8, 128), lambda n: (n, 0, 0, 0)),
            pl.BlockSpec((1, 128), lambda n: (0, 0)),
            pl.BlockSpec((1, 128), lambda n: (0, 0)),
            pl.BlockSpec((9, 128, 128), lambda n: (0, 0, 0)),
            pl.BlockSpec((1, 128), lambda n: (0, 0)),
            pl.BlockSpec((9, 128, 128), lambda n: (0, 0, 0)),
            pl.BlockSpec((1, 128), lambda n: (0, 0)),
            pl.BlockSpec((9, 128, 128), lambda n: (0, 0, 0)),
            pl.BlockSpec((1, 128), lambda n: (0, 0)),
            pl.BlockSpec((128, 128), lambda n: (0, 0)),
            pl.BlockSpec((1, 128), lambda n: (0, 0)),
        ],
        out_specs=pl.BlockSpec((1, 1, 128), lambda n: (n, 0, 0)),
        out_shape=jax.ShapeDtypeStruct((N, 1, 128), f32),
        scratch_shapes=[pltpu.VMEM((10, 10, 128), f32),
                        pltpu.VMEM((6, 6, 128), f32),
                        pltpu.VMEM((4, 4, 128), f32)],
        compiler_params=_cparams(("parallel",)),
    )(a2, sc2, sh2,
      fmtl0_w.reshape(9, 128, 128).astype(bf16), fmtl0_b.reshape(1, 128).astype(f32),
      fmtl1_w.reshape(9, 128, 128).astype(bf16), fmtl1_b.reshape(1, 128).astype(f32),
      fmtl2_w.reshape(9, 128, 128).astype(bf16), fmtl2_b.reshape(1, 128).astype(f32),
      fc_w.reshape(128, 128).astype(bf16), fc_b.reshape(1, 128).astype(f32))

    zb = z.reshape(N, 128).astype(bf16)
    bws = (block0_w, block1_w, block2_w, block3_w)
    wsq = jnp.stack([jnp.sum(w * w, axis=(0, 1)) for w in bws])
    wsq = jnp.concatenate([wsq, jnp.zeros((1, 128, 128), f32)], axis=0)

    styles5, d5 = pl.pallas_call(
        _k5,
        grid=(5,),
        in_specs=[
            pl.BlockSpec((N, 128), lambda g: (0, 0)),
            pl.BlockSpec((1, 128, 128), lambda g: (g, 0, 0)),
            pl.BlockSpec((1, 1, 128), lambda g: (g, 0, 0)),
            pl.BlockSpec((1, 128, 128), lambda g: (g, 0, 0)),
        ],
        out_specs=(pl.BlockSpec((1, N, 128), lambda g: (g, 0, 0)),
                   pl.BlockSpec((1, N, 128), lambda g: (g, 0, 0))),
        out_shape=(jax.ShapeDtypeStruct((5, N, 128), f32),
                   jax.ShapeDtypeStruct((5, N, 128), f32)),
        compiler_params=_cparams(("parallel",)),
    )(zb, style_w.astype(bf16), style_b.astype(f32), wsq)

    stp = jnp.transpose(styles5, (1, 0, 2))
    dp = jnp.transpose(d5[:4], (1, 0, 2))

    bias_dec = jnp.stack([block0_b, block1_b, block2_b, block3_b])
    bias_dec = bias_dec.reshape(4, 1, 128).astype(f32)

    out = pl.pallas_call(
        _k6,
        grid=(N,),
        in_specs=[
            pl.BlockSpec((8, 8, 128), lambda n: (0, 0, 0)),
            pl.BlockSpec((1, 5, 128), lambda n: (n, 0, 0)),
            pl.BlockSpec((1, 4, 128), lambda n: (n, 0, 0)),
            pl.BlockSpec((9, 128, 128), lambda n: (0, 0, 0)),
            pl.BlockSpec((9, 128, 128), lambda n: (0, 0, 0)),
            pl.BlockSpec((9, 128, 128), lambda n: (0, 0, 0)),
            pl.BlockSpec((9, 128, 128), lambda n: (0, 0, 0)),
            pl.BlockSpec((4, 1, 128), lambda n: (0, 0, 0)),
            pl.BlockSpec((128, 128), lambda n: (0, 0)),
            pl.BlockSpec((1, 128), lambda n: (0, 0)),
        ],
        out_specs=pl.BlockSpec((1, 3, 64, 64), lambda n: (n, 0, 0, 0)),
        out_shape=jax.ShapeDtypeStruct((N, 3, 64, 64), f32),
        scratch_shapes=[pltpu.VMEM((10, 10, 128), bf16),
                        pltpu.VMEM((18, 18, 128), f32),
                        pltpu.VMEM((18, 18, 128), bf16),
                        pltpu.VMEM((34, 34, 128), f32),
                        pltpu.VMEM((34, 34, 128), bf16),
                        pltpu.VMEM((66, 66, 128), f32),
                        pltpu.VMEM((66, 66, 128), bf16)],
        compiler_params=_cparams(("parallel",)),
    )(const, stp, dp,
      block0_w.reshape(9, 128, 128).astype(bf16),
      block1_w.reshape(9, 128, 128).astype(bf16),
      block2_w.reshape(9, 128, 128).astype(bf16),
      block3_w.reshape(9, 128, 128).astype(bf16),
      bias_dec,
      torgb_w.reshape(128, 128).astype(bf16),
      torgb_b.reshape(1, 128).astype(f32))

    return out


# speedup vs baseline: 13.5059x; 1.3956x over previous
import functools

import jax
import jax.numpy as jnp
from jax.experimental import pallas as pl
from jax.experimental.pallas import tpu as pltpu

_VMEM = 64 * 1024 * 1024
_SQRT2 = 1.4142135623730951
_TAPS = tuple((kh, kw) for kh in range(3) for kw in range(3))


def _mish(y):
    t = jnp.exp(jnp.minimum(y, 20.0))
    u = 1.0 + t
    n = u * u
    return y * (n - 1.0) * pl.reciprocal(n + 1.0, approx=True)


def _lrelu(y):
    return jnp.where(y >= 0.0, y, 0.2 * y) * _SQRT2


def _store_padded(ref, h, hw):
    z = jnp.zeros((1, hw + 2, h.shape[-1]), h.dtype)
    ref[pl.ds(0, 1), pl.ds(0, hw + 2), :] = z
    ref[pl.ds(hw + 1, 1), pl.ds(0, hw + 2), :] = z
    zc = jnp.zeros((hw, 1, h.shape[-1]), h.dtype)
    ref[pl.ds(1, hw), pl.ds(0, 1), :] = zc
    ref[pl.ds(1, hw), pl.ds(hw + 1, 1), :] = zc
    ref[pl.ds(1, hw), pl.ds(1, hw), :] = h


def _store_padded_up2(ref, xm, hw_out):
    hw_in = hw_out // 2
    z = jnp.zeros((1, hw_out + 2, xm.shape[-1]), xm.dtype)
    ref[pl.ds(0, 1), pl.ds(0, hw_out + 2), :] = z
    ref[pl.ds(hw_out + 1, 1), pl.ds(0, hw_out + 2), :] = z
    zc = jnp.zeros((hw_out, 1, xm.shape[-1]), xm.dtype)
    ref[pl.ds(1, hw_out), pl.ds(0, 1), :] = zc
    ref[pl.ds(1, hw_out), pl.ds(hw_out + 1, 1), :] = zc
    for a in (0, 1):
        for b in (0, 1):
            ref[pl.Slice(1 + a, hw_in, 2), pl.Slice(1 + b, hw_in, 2), :] = xm


def _conv_s1(pref, wt, ho, wo):
    m = ho * wo
    acc = jnp.zeros((m, wt.shape[-1]), jnp.float32)
    for t, (kh, kw) in enumerate(_TAPS):
        xs = pref[pl.ds(kh, ho), pl.ds(kw, wo), :]
        acc = acc + jnp.dot(xs.reshape(m, xs.shape[-1]), wt[t],
                            preferred_element_type=jnp.float32)
    return acc


def _conv_s2(pref, wt, ho, wo):
    m = ho * wo
    acc = jnp.zeros((m, wt.shape[-1]), jnp.float32)
    for t, (kh, kw) in enumerate(_TAPS):
        xs = pref[pl.Slice(kh, ho, 2), pl.Slice(kw, wo, 2), :]
        xs = xs.astype(jnp.bfloat16).reshape(m, xs.shape[-1])
        acc = acc + jnp.dot(xs, wt[t], preferred_element_type=jnp.float32)
    return acc


def _k1(x_ref, sw_ref, sb_ref, dw_ref, db_ref, y_ref, st_ref, xp_ref, pad_ref):
    xb = jnp.transpose(x_ref[0].astype(jnp.bfloat16), (1, 2, 0))
    _store_padded(xp_ref, xb, 64)
    acc = jnp.zeros((4096, 128), jnp.float32)
    for t, (kh, kw) in enumerate(_TAPS):
        xs = xp_ref[pl.ds(kh, 64), pl.ds(kw, 64), :].reshape(4096, 3)
        acc = acc + jnp.dot(xs, sw_ref[t], preferred_element_type=jnp.float32)
    h = _mish(acc + sb_ref[...]).astype(jnp.bfloat16).reshape(64, 64, 128)
    _store_padded(pad_ref, h.astype(jnp.float32), 64)
    y = _conv_s2(pad_ref, dw_ref[...], 32, 32) + db_ref[...]
    st_ref[0, 0:1, :] = jnp.sum(y, axis=0, keepdims=True)
    st_ref[0, 1:2, :] = jnp.sum(y * y, axis=0, keepdims=True)
    y_ref[0] = y.reshape(32, 32, 128).astype(jnp.bfloat16)


def _k23(a_ref, sc_ref, sh_ref, w_ref, b_ref, y_ref, st_ref, pad_ref, *, hi):
    ho = hi // 2
    h = _mish(a_ref[0].astype(jnp.float32) * sc_ref[...] + sh_ref[...])
    _store_padded(pad_ref, h.astype(jnp.bfloat16).astype(jnp.float32), hi)
    y = _conv_s2(pad_ref, w_ref[...], ho, ho) + b_ref[...]
    st_ref[0, 0:1, :] = jnp.sum(y, axis=0, keepdims=True)
    st_ref[0, 1:2, :] = jnp.sum(y * y, axis=0, keepdims=True)
    y_ref[0] = y.reshape(ho, ho, 128).astype(jnp.bfloat16)


def _k4(a_ref, sc_ref, sh_ref, w0_ref, b0_ref, w1_ref, b1_ref,
        w2_ref, b2_ref, fw_ref, fb_ref, z_ref, p0_ref, p1_ref, p2_ref):
    h = _mish(a_ref[0].astype(jnp.float32) * sc_ref[...] + sh_ref[...])
    _store_padded(p0_ref, h.astype(jnp.bfloat16).astype(jnp.float32), 8)
    h = _mish(_conv_s2(p0_ref, w0_ref[...], 4, 4) + b0_ref[...])
    h = h.astype(jnp.bfloat16).astype(jnp.float32).reshape(4, 4, 128)
    _store_padded(p1_ref, h, 4)
    h = _mish(_conv_s2(p1_ref, w1_ref[...], 2, 2) + b1_ref[...])
    h = h.astype(jnp.bfloat16).astype(jnp.float32).reshape(2, 2, 128)
    _store_padded(p2_ref, h, 2)
    h = _mish(_conv_s2(p2_ref, w2_ref[...], 1, 1) + b2_ref[...])
    z_ref[0] = jnp.dot(h.astype(jnp.bfloat16), fw_ref[...],
                       preferred_element_type=jnp.float32) + fb_ref[...]


def _k5(z_ref, sw_ref, sb_ref, q_ref, s_ref, d_ref):
    st = jnp.dot(z_ref[...], sw_ref[0],
                 preferred_element_type=jnp.float32) + sb_ref[0]
    s_ref[0] = st
    d_ref[0] = jax.lax.rsqrt(
        jnp.dot(st * st, q_ref[0], preferred_element_type=jnp.float32) + 1e-8)


def _k6(c_ref, st_ref, d_ref, wa_ref, wb_ref, wc_ref, wd_ref,
        bias_ref, tw_ref, tb_ref, o_ref, pa_ref,
        qb_ref, pb_ref, qc_ref, pc_ref, qd_ref, pd_ref):
    st = st_ref[0]
    dm = d_ref[0]
    x = c_ref[...].astype(jnp.bfloat16).astype(jnp.float32) * st[0:1, :]
    _store_padded(pa_ref, x.astype(jnp.bfloat16), 8)
    ws = (wa_ref, wb_ref, wc_ref, wd_ref)
    pads = (pa_ref, pb_ref, pc_ref, pd_ref)
    ups = (None, qb_ref, qc_ref, qd_ref)
    for g, hw in enumerate((8, 16, 32, 64)):
        if g > 0:
            xm = (x.astype(jnp.float32) * st[g:g + 1, :]).astype(jnp.bfloat16)
            _store_padded_up2(ups[g], xm.astype(jnp.float32), hw)
            pads[g][...] = ups[g][...].astype(jnp.bfloat16)
        y = _conv_s1(pads[g], ws[g][...], hw, hw)
        y = y * dm[g:g + 1, :] + bias_ref[g]
        x = _lrelu(y).astype(jnp.bfloat16).reshape(hw, hw, 128)
    xm = (x.astype(jnp.float32) * st[4:5, :]).astype(jnp.bfloat16)
    y = jnp.dot(xm.reshape(4096, 128), tw_ref[...],
                preferred_element_type=jnp.float32) + tb_ref[...]
    for s in range(4):
        y3 = y[:, 3 * s:3 * s + 3].astype(jnp.bfloat16).astype(jnp.float32)
        o_ref[s] = jnp.transpose(y3.reshape(64, 64, 3), (2, 0, 1))


def _bn_coeffs(stats, gamma, beta, cnt):
    tot = jnp.sum(stats, axis=0)
    mean = tot[0] / cnt
    var = jnp.maximum(tot[1] / cnt - mean * mean, 0.0)
    inv = jax.lax.rsqrt(var + 1e-5)
    sc = (gamma * inv).astype(jnp.float32)
    sh = (beta - mean * sc).astype(jnp.float32)
    return sc.reshape(1, 128), sh.reshape(1, 128)


def _cparams(sem):
    return pltpu.CompilerParams(dimension_semantics=sem,
                                vmem_limit_bytes=_VMEM)


def kernel(x, stem_w, stem_b, down0_w, down0_b, down0_gamma, down0_beta,
           down1_w, down1_b, down1_gamma, down1_beta,
           down2_w, down2_b, down2_gamma, down2_beta,
           fmtl0_w, fmtl0_b, fmtl1_w, fmtl1_b, fmtl2_w, fmtl2_b,
           fc_w, fc_b, const,
           block0_w, block0_b, block1_w, block1_b,
           block2_w, block2_b, block3_w, block3_b,
           torgb_w, torgb_b, style_w, style_b):
    f32, bf16 = jnp.float32, jnp.bfloat16
    N = x.shape[0]

    sw = stem_w.reshape(9, 3, 128).astype(bf16)
    sb = stem_b.reshape(1, 128).astype(f32)
    d0w = down0_w.reshape(9, 128, 128).astype(bf16)
    d0b = down0_b.reshape(1, 128).astype(f32)

    a0, s0 = pl.pallas_call(
        _k1,
        grid=(N,),
        in_specs=[
            pl.BlockSpec((1, 3, 64, 64), lambda n: (n, 0, 0, 0)),
            pl.BlockSpec((9, 3, 128), lambda n: (0, 0, 0)),
            pl.BlockSpec((1, 128), lambda n: (0, 0)),
            pl.BlockSpec((9, 128, 128), lambda n: (0, 0, 0)),
            pl.BlockSpec((1, 128), lambda n: (0, 0)),
        ],
        out_specs=(pl.BlockSpec((1, 32, 32, 128), lambda n: (n, 0, 0, 0)),
                   pl.BlockSpec((1, 2, 128), lambda n: (n, 0, 0))),
        out_shape=(jax.ShapeDtypeStruct((N, 32, 32, 128), bf16),
                   jax.ShapeDtypeStruct((N, 2, 128), f32)),
        scratch_shapes=[pltpu.VMEM((66, 66, 3), bf16),
                        pltpu.VMEM((66, 66, 128), f32)],
        compiler_params=_cparams(("parallel",)),
    )(x, sw, sb, d0w, d0b)

    def down_call(a, stats, gamma, beta, w, b, hi):
        cnt = float(N * hi * hi)
        sc, sh = _bn_coeffs(stats, gamma, beta, cnt)
        ho = hi // 2
        return pl.pallas_call(
            functools.partial(_k23, hi=hi),
            grid=(N,),
            in_specs=[
                pl.BlockSpec((1, hi, hi, 128), lambda n: (n, 0, 0, 0)),
                pl.BlockSpec((1, 128), lambda n: (0, 0)),
                pl.BlockSpec((1, 128), lambda n: (0, 0)),
                pl.BlockSpec((9, 128, 128), lambda n: (0, 0, 0)),
                pl.BlockSpec((1, 128), lambda n: (0, 0)),
            ],
            out_specs=(pl.BlockSpec((1, ho, ho, 128), lambda n: (n, 0, 0, 0)),
                       pl.BlockSpec((1, 2, 128), lambda n: (n, 0, 0))),
            out_shape=(jax.ShapeDtypeStruct((N, ho, ho, 128), bf16),
                       jax.ShapeDtypeStruct((N, 2, 128), f32)),
            scratch_shapes=[pltpu.VMEM((hi + 2, hi + 2, 128), f32)],
            compiler_params=_cparams(("parallel",)),
        )(a, sc, sh, w.reshape(9, 128, 128).astype(bf16),
          b.reshape(1, 128).astype(f32))

    a1, s1 = down_call(a0, s0, down0_gamma, down0_beta, down1_w, down1_b, 32)
    a2, s2 = down_call(a1, s1, down1_gamma, down1_beta, down2_w, down2_b, 16)

    sc2, sh2 = _bn_coeffs(s2, down2_gamma, down2_beta, float(N * 8 * 8))
    z = pl.pallas_call(
        _k4,
        grid=(N,),
        in_specs=[
            pl.BlockSpec((1, 8, 8, 128), lambda n: (n, 0, 0, 0)),
            pl.BlockSpec((1, 128), lambda n: (0, 0)),
            pl.BlockSpec((1, 128), lambda n: (0, 0)),
            pl.BlockSpec((9, 128, 128), lambda n: (0, 0, 0)),
            pl.BlockSpec((1, 128), lambda n: (0, 0)),
            pl.BlockSpec((9, 128, 128), lambda n: (0, 0, 0)),
            pl.BlockSpec((1, 128), lambda n: (0, 0)),
            pl.BlockSpec((9, 128, 128), lambda n: (0, 0, 0)),
            pl.BlockSpec((1, 128), lambda n: (0, 0)),
            pl.BlockSpec((128, 128), lambda n: (0, 0)),
            pl.BlockSpec((1, 128), lambda n: (0, 0)),
        ],
        out_specs=pl.BlockSpec((1, 1, 128), lambda n: (n, 0, 0)),
        out_shape=jax.ShapeDtypeStruct((N, 1, 128), f32),
        scratch_shapes=[pltpu.VMEM((10, 10, 128), f32),
                        pltpu.VMEM((6, 6, 128), f32),
                        pltpu.VMEM((4, 4, 128), f32)],
        compiler_params=_cparams(("parallel",)),
    )(a2, sc2, sh2,
      fmtl0_w.reshape(9, 128, 128).astype(bf16), fmtl0_b.reshape(1, 128).astype(f32),
      fmtl1_w.reshape(9, 128, 128).astype(bf16), fmtl1_b.reshape(1, 128).astype(f32),
      fmtl2_w.reshape(9, 128, 128).astype(bf16), fmtl2_b.reshape(1, 128).astype(f32),
      fc_w.reshape(128, 128).astype(bf16), fc_b.reshape(1, 128).astype(f32))

    zb = z.reshape(N, 128).astype(bf16)
    bws = (block0_w, block1_w, block2_w, block3_w)
    wsq = jnp.stack([jnp.sum(w * w, axis=(0, 1)) for w in bws])
    wsq = jnp.concatenate([wsq, jnp.zeros((1, 128, 128), f32)], axis=0)

    styles5, d5 = pl.pallas_call(
        _k5,
        grid=(5,),
        in_specs=[
            pl.BlockSpec((N, 128), lambda g: (0, 0)),
            pl.BlockSpec((1, 128, 128), lambda g: (g, 0, 0)),
            pl.BlockSpec((1, 1, 128), lambda g: (g, 0, 0)),
            pl.BlockSpec((1, 128, 128), lambda g: (g, 0, 0)),
        ],
        out_specs=(pl.BlockSpec((1, N, 128), lambda g: (g, 0, 0)),
                   pl.BlockSpec((1, N, 128), lambda g: (g, 0, 0))),
        out_shape=(jax.ShapeDtypeStruct((5, N, 128), f32),
                   jax.ShapeDtypeStruct((5, N, 128), f32)),
        compiler_params=_cparams(("parallel",)),
    )(zb, style_w.astype(bf16), style_b.astype(f32), wsq)

    cinr = (32, 32, 32, 16, 8)
    coutr = (32, 32, 16, 8)
    G4 = N // 4

    stp = jnp.zeros((G4, 5, 128), f32)
    for g in range(5):
        sg = styles5[g][:, :cinr[g]].reshape(G4, 4 * cinr[g])
        stp = stp.at[:, g, :4 * cinr[g]].set(sg)
    dp = jnp.zeros((G4, 4, 128), f32)
    for g in range(4):
        dg = d5[g][:, :coutr[g]].reshape(G4, 4 * coutr[g])
        dp = dp.at[:, g, :4 * coutr[g]].set(dg)

    def pack_w(w, ci, co):
        wr = w.reshape(9, 128, 128)[:, :ci, :co]
        w4 = jax.vmap(lambda m: jnp.kron(jnp.eye(4, dtype=f32), m))(wr)
        out = jnp.zeros((9, 128, 128), f32)
        return out.at[:, :4 * ci, :4 * co].set(w4).astype(bf16)

    w4s = [pack_w(w, cinr[g], coutr[g])
           for g, w in enumerate((block0_w, block1_w, block2_w, block3_w))]
    bias_dec = jnp.zeros((4, 1, 128), f32)
    for g, b in enumerate((block0_b, block1_b, block2_b, block3_b)):
        bias_dec = bias_dec.at[g, 0, :4 * coutr[g]].set(
            jnp.tile(b[:coutr[g]], 4))
    tw4 = jnp.zeros((128, 128), f32).at[:32, :12].set(
        jnp.kron(jnp.eye(4, dtype=f32), torgb_w.reshape(128, 128)[:8, :3]))
    tb4 = jnp.zeros((1, 128), f32).at[0, :12].set(jnp.tile(torgb_b[:3], 4))
    const4 = jnp.tile(const[:, :, :32], (1, 1, 4))

    out = pl.pallas_call(
        _k6,
        grid=(G4,),
        in_specs=[
            pl.BlockSpec((8, 8, 128), lambda n: (0, 0, 0)),
            pl.BlockSpec((1, 5, 128), lambda n: (n, 0, 0)),
            pl.BlockSpec((1, 4, 128), lambda n: (n, 0, 0)),
            pl.BlockSpec((9, 128, 128), lambda n: (0, 0, 0)),
            pl.BlockSpec((9, 128, 128), lambda n: (0, 0, 0)),
            pl.BlockSpec((9, 128, 128), lambda n: (0, 0, 0)),
            pl.BlockSpec((9, 128, 128), lambda n: (0, 0, 0)),
            pl.BlockSpec((4, 1, 128), lambda n: (0, 0, 0)),
            pl.BlockSpec((128, 128), lambda n: (0, 0)),
            pl.BlockSpec((1, 128), lambda n: (0, 0)),
        ],
        out_specs=pl.BlockSpec((4, 3, 64, 64), lambda n: (n, 0, 0, 0)),
        out_shape=jax.ShapeDtypeStruct((N, 3, 64, 64), f32),
        scratch_shapes=[pltpu.VMEM((10, 10, 128), bf16),
                        pltpu.VMEM((18, 18, 128), f32),
                        pltpu.VMEM((18, 18, 128), bf16),
                        pltpu.VMEM((34, 34, 128), f32),
                        pltpu.VMEM((34, 34, 128), bf16),
                        pltpu.VMEM((66, 66, 128), f32),
                        pltpu.VMEM((66, 66, 128), bf16)],
        compiler_params=_cparams(("parallel",)),
    )(const4, stp, dp, w4s[0], w4s[1], w4s[2], w4s[3],
      bias_dec, tw4.astype(bf16), tb4)

    return out
